# baseline (speedup 1.0000x reference)
import functools

import jax
import jax.numpy as jnp
from jax import lax
from jax.experimental import pallas as pl
from jax.experimental.pallas import tpu as pltpu
from jax.experimental.pallas import tpu_sc as plsc

D = 64
B = 4096
NNEG = 128
L = 16
NC, NS = 2, 16
NW = NC * NS
BW = B // NW
DCH = D // L
JG = NNEG // L


def _neg_compute(i, nbuf, qrow, nout, lanes):
  accs = [jnp.zeros((L,), jnp.float32) for _ in range(JG)]
  qv = [qrow[i, pl.ds(c * L, L)] for c in range(DCH)]
  for d in range(D):
    qs = qv[d // L][d % L]
    di = jnp.full((L,), d, jnp.int32)
    for jg in range(JG):
      col = plsc.load_gather(nbuf, [lanes + jg * L, di])
      accs[jg] = accs[jg] + col * qs
  for jg in range(JG):
    nout[i, pl.ds(jg * L, L)] = accs[jg]


def _body(entity_hbm, relation_hbm, head_hbm, rel_hbm, ans_hbm, neg_hbm,
          scores_hbm, negsc_hbm,
          hidx, ridx, aidx, nidx, hrow, rrow, arow, qrow,
          nega, negb, scr, nout, semg, sema, semb):
  wid = lax.axis_index("s") * NC + lax.axis_index("c")
  base = wid * BW
  lanes = lax.iota(jnp.int32, L)

  pltpu.sync_copy(head_hbm.at[pl.ds(base, BW)], hidx)
  pltpu.sync_copy(rel_hbm.at[pl.ds(base, BW)], ridx)
  pltpu.sync_copy(ans_hbm.at[pl.ds(base, BW)], aidx)
  pltpu.sync_copy(neg_hbm.at[pl.ds(base, BW)], nidx)

  pltpu.async_copy(entity_hbm.at[nidx.at[0]], nega, sema)
  cph = pltpu.async_copy(entity_hbm.at[hidx], hrow, semg)
  cpr = pltpu.async_copy(relation_hbm.at[ridx], rrow, semg)
  cpa = pltpu.async_copy(entity_hbm.at[aidx], arow, semg)
  cph.wait()
  cpr.wait()
  cpa.wait()

  def qbody(i, c):
    for ch in range(DCH):
      sl = pl.ds(ch * L, L)
      qrow[i, sl] = hrow[i, sl] * rrow[i, sl]
    return c
  lax.fori_loop(0, BW, qbody, 0)

  def pbody(g, c):
    qi = lanes + g * L
    acc = jnp.zeros((L,), jnp.float32)
    for d in range(D):
      di = jnp.full((L,), d, jnp.int32)
      acc = acc + plsc.load_gather(qrow, [qi, di]) * plsc.load_gather(arow, [qi, di])
    scr[pl.ds(pl.multiple_of(g * L, L), L)] = acc
    return c
  lax.fori_loop(0, BW // L, pbody, 0)

  def nbody(i2, c):
    qa = 2 * i2
    qb = qa + 1
    cpb = pltpu.async_copy(entity_hbm.at[nidx.at[qb]], negb, semb)
    pltpu.make_async_copy(entity_hbm.at[nidx.at[qa]], nega, sema).wait()
    _neg_compute(qa, nega, qrow, nout, lanes)

    @pl.when(i2 < BW // 2 - 1)
    def _():
      pltpu.async_copy(entity_hbm.at[nidx.at[qa + 2]], nega, sema)

    cpb.wait()
    _neg_compute(qb, negb, qrow, nout, lanes)
    return c
  lax.fori_loop(0, BW // 2, nbody, 0)

  pltpu.sync_copy(scr, scores_hbm.at[pl.ds(base, BW)])
  pltpu.sync_copy(nout, negsc_hbm.at[pl.ds(base, BW)])


@jax.jit
def kernel(entity_embedding, relation_embedding, head_idx, rel_idx, answer_idx, neg_idx):
  mesh = plsc.VectorSubcoreMesh(core_axis_name="c", subcore_axis_name="s")
  run = pl.kernel(
      _body,
      out_type=(
          jax.ShapeDtypeStruct((B,), jnp.float32),
          jax.ShapeDtypeStruct((B, NNEG), jnp.float32),
      ),
      mesh=mesh,
      compiler_params=pltpu.CompilerParams(
          needs_layout_passes=False, use_tc_tiling_on_sc=False),
      scratch_types=[
          pltpu.VMEM((BW,), jnp.int32),
          pltpu.VMEM((BW,), jnp.int32),
          pltpu.VMEM((BW,), jnp.int32),
          pltpu.VMEM((BW, NNEG), jnp.int32),
          pltpu.VMEM((BW, D), jnp.float32),
          pltpu.VMEM((BW, D), jnp.float32),
          pltpu.VMEM((BW, D), jnp.float32),
          pltpu.VMEM((BW, D), jnp.float32),
          pltpu.VMEM((NNEG, D), jnp.float32),
          pltpu.VMEM((NNEG, D), jnp.float32),
          pltpu.VMEM((BW,), jnp.float32),
          pltpu.VMEM((BW, NNEG), jnp.float32),
          pltpu.SemaphoreType.DMA,
          pltpu.SemaphoreType.DMA,
          pltpu.SemaphoreType.DMA,
      ],
  )
  return run(entity_embedding, relation_embedding, head_idx, rel_idx,
             answer_idx, neg_idx)


# speedup vs baseline: 1.7132x; 1.7132x over previous
import functools

import jax
import jax.numpy as jnp
from jax import lax
from jax.experimental import pallas as pl
from jax.experimental.pallas import tpu as pltpu
from jax.experimental.pallas import tpu_sc as plsc

D = 64
B = 4096
NNEG = 128
L = 16
NC, NS = 2, 16
NW = NC * NS
BW = B // NW
DCH = D // L
JG = NNEG // L


def _dot16(row_of, qv, lanes):
  outv = jnp.zeros((L,), jnp.float32)
  for t in range(L):
    chunks = row_of(t)
    p = qv[0] * chunks[0]
    for c in range(1, DCH):
      p = p + qv[c] * chunks[c]
    outv = jnp.where(lanes == t, jnp.sum(p), outv)
  return outv


def _neg_compute(i, nbuf, qrow, nout, lanes):
  qv = [qrow[i, pl.ds(c * L, L)] for c in range(DCH)]
  for jg in range(JG):
    row_of = lambda t: [nbuf[jg * L + t, pl.ds(c * L, L)] for c in range(DCH)]
    nout[i, pl.ds(jg * L, L)] = _dot16(row_of, qv, lanes)


def _body(entity_hbm, relation_hbm, head_hbm, rel_hbm, ans_hbm, neg_hbm,
          scores_hbm, negsc_hbm,
          hidx, ridx, aidx, nidx, hrow, rrow, arow, qrow,
          nega, negb, scr, nout, semg, sema, semb):
  wid = lax.axis_index("s") * NC + lax.axis_index("c")
  base = wid * BW
  lanes = lax.iota(jnp.int32, L)

  pltpu.sync_copy(head_hbm.at[pl.ds(base, BW)], hidx)
  pltpu.sync_copy(rel_hbm.at[pl.ds(base, BW)], ridx)
  pltpu.sync_copy(ans_hbm.at[pl.ds(base, BW)], aidx)
  pltpu.sync_copy(neg_hbm.at[pl.ds(base, BW)], nidx)

  pltpu.async_copy(entity_hbm.at[nidx.at[0]], nega, sema)
  cph = pltpu.async_copy(entity_hbm.at[hidx], hrow, semg)
  cpr = pltpu.async_copy(relation_hbm.at[ridx], rrow, semg)
  cpa = pltpu.async_copy(entity_hbm.at[aidx], arow, semg)
  cph.wait()
  cpr.wait()
  cpa.wait()

  def qbody(i, c):
    for ch in range(DCH):
      sl = pl.ds(ch * L, L)
      qrow[i, sl] = hrow[i, sl] * rrow[i, sl]
    return c
  lax.fori_loop(0, BW, qbody, 0)

  def pbody(g, c):
    outv = jnp.zeros((L,), jnp.float32)
    for t in range(L):
      i = g * L + t
      p = qrow[i, pl.ds(0, L)] * arow[i, pl.ds(0, L)]
      for ch in range(1, DCH):
        sl = pl.ds(ch * L, L)
        p = p + qrow[i, sl] * arow[i, sl]
      outv = jnp.where(lanes == t, jnp.sum(p), outv)
    scr[pl.ds(pl.multiple_of(g * L, L), L)] = outv
    return c
  lax.fori_loop(0, BW // L, pbody, 0)

  def nbody(i2, c):
    qa = 2 * i2
    qb = qa + 1
    cpb = pltpu.async_copy(entity_hbm.at[nidx.at[qb]], negb, semb)
    pltpu.make_async_copy(entity_hbm.at[nidx.at[qa]], nega, sema).wait()
    _neg_compute(qa, nega, qrow, nout, lanes)

    @pl.when(i2 < BW // 2 - 1)
    def _():
      pltpu.async_copy(entity_hbm.at[nidx.at[qa + 2]], nega, sema)

    cpb.wait()
    _neg_compute(qb, negb, qrow, nout, lanes)
    return c
  lax.fori_loop(0, BW // 2, nbody, 0)

  pltpu.sync_copy(scr, scores_hbm.at[pl.ds(base, BW)])
  pltpu.sync_copy(nout, negsc_hbm.at[pl.ds(base, BW)])


@jax.jit
def kernel(entity_embedding, relation_embedding, head_idx, rel_idx, answer_idx, neg_idx):
  mesh = plsc.VectorSubcoreMesh(core_axis_name="c", subcore_axis_name="s")
  run = pl.kernel(
      _body,
      out_type=(
          jax.ShapeDtypeStruct((B,), jnp.float32),
          jax.ShapeDtypeStruct((B, NNEG), jnp.float32),
      ),
      mesh=mesh,
      compiler_params=pltpu.CompilerParams(
          needs_layout_passes=False, use_tc_tiling_on_sc=False),
      scratch_types=[
          pltpu.VMEM((BW,), jnp.int32),
          pltpu.VMEM((BW,), jnp.int32),
          pltpu.VMEM((BW,), jnp.int32),
          pltpu.VMEM((BW, NNEG), jnp.int32),
          pltpu.VMEM((BW, D), jnp.float32),
          pltpu.VMEM((BW, D), jnp.float32),
          pltpu.VMEM((BW, D), jnp.float32),
          pltpu.VMEM((BW, D), jnp.float32),
          pltpu.VMEM((NNEG, D), jnp.float32),
          pltpu.VMEM((NNEG, D), jnp.float32),
          pltpu.VMEM((BW,), jnp.float32),
          pltpu.VMEM((BW, NNEG), jnp.float32),
          pltpu.SemaphoreType.DMA,
          pltpu.SemaphoreType.DMA,
          pltpu.SemaphoreType.DMA,
      ],
  )
  return run(entity_embedding, relation_embedding, head_idx, rel_idx,
             answer_idx, neg_idx)
